# Initial kernel scaffold; baseline (speedup 1.0000x reference)
import jax
import jax.numpy as jnp
from jax.experimental import pallas as pl


def kernel(x_s, x_t, term_walk_index_s, term_walk_index_t, x_s_batch, x_t_batch, y, params):
    raise NotImplementedError("write your pallas kernel here")


# speedup vs baseline: 1.0519x; 1.0519x over previous
import functools

import jax
import jax.numpy as jnp
from jax import lax
from jax.experimental import pallas as pl
from jax.experimental.pallas import tpu as pltpu
from jax.experimental.pallas import tpu_sc as plsc

N_NODES = 10000
N_WALKS = 160000
N_IN = 128
D = 256
D3 = 3 * D
LAYERS = 2
N_GRAPH = 64
EPS = 1e-5

NC = 2
NS = 16
NW = NC * NS

_f32 = jnp.float32


def _embed_body(x_ref, emb_ref, o_ref):
    x = x_ref[...]
    mx = jnp.max(x, axis=1, keepdims=True)
    eq = x == mx
    it = lax.broadcasted_iota(jnp.int32, x.shape, 1)
    first_idx = jnp.min(jnp.where(eq, it, N_IN), axis=1, keepdims=True)
    oh = (it == first_idx).astype(_f32)
    o_ref[...] = jnp.dot(oh, emb_ref[...], preferred_element_type=_f32)


def _embed(x, emb):
    return pl.pallas_call(
        _embed_body,
        out_shape=jax.ShapeDtypeStruct((N_NODES, D), _f32),
    )(x, emb)


_AM_BLK = 2000
_AM_G = N_NODES // _AM_BLK


def _amat_body(h_ref, w_ref, cnt_ref, a_ref, mean_ref):
    i = pl.program_id(0)
    a = jnp.dot(h_ref[...], w_ref[...], preferred_element_type=_f32)
    a_ref[...] = a
    m = jnp.zeros((1, D3), _f32)
    for k in range(3):
        ck = cnt_ref[:, k:k + 1]
        m = m + lax.dot_general(ck, a[:, k * D3:(k + 1) * D3],
                                (((0,), (0,)), ((), ())),
                                preferred_element_type=_f32)

    @pl.when(i == 0)
    def _():
        mean_ref[...] = jnp.zeros((1, D3), _f32)

    mean_ref[...] += m / N_WALKS


def _amat(h, wcat, cnt3):
    return pl.pallas_call(
        _amat_body,
        grid=(_AM_G,),
        in_specs=[pl.BlockSpec((_AM_BLK, D), lambda i: (i, 0)),
                  pl.BlockSpec((D, 3 * D3), lambda i: (0, 0)),
                  pl.BlockSpec((_AM_BLK, 3), lambda i: (i, 0))],
        out_specs=[pl.BlockSpec((_AM_BLK, 3 * D3), lambda i: (i, 0)),
                   pl.BlockSpec((1, D3), lambda i: (0, 0))],
        out_shape=[jax.ShapeDtypeStruct((N_NODES, 3 * D3), _f32),
                   jax.ShapeDtypeStruct((1, D3), _f32)],
    )(h, wcat, cnt3)


def _fin_body(ssq_ref, mean_ref, g_ref, be_ref, a_ref, b_ref):
    ssq = jnp.sum(ssq_ref[...], axis=0, keepdims=True)
    mean = mean_ref[...]
    var = ssq / N_WALKS - mean * mean
    alpha = g_ref[...] * lax.rsqrt(var + EPS)
    a_ref[...] = alpha
    b_ref[...] = be_ref[...] - mean * alpha


def _fin(ssqp, mean, g, be):
    return pl.pallas_call(
        _fin_body,
        out_shape=[jax.ShapeDtypeStruct((1, D3), _f32),
                   jax.ShapeDtypeStruct((1, D3), _f32)],
    )(ssqp, mean, g, be)


_FTW_BLK = 2000
_FTW_G = N_NODES // _FTW_BLK


def _ftwa_body(acc_ref, cnt_ref, w_ref, z_ref, st_ref):
    i = pl.program_id(0)
    msum = jnp.zeros((_FTW_BLK, D), _f32)
    for j in range(3):
        aj = jnp.concatenate([acc_ref[j, 0], acc_ref[j, 1]], axis=1)
        msum = msum + aj / jnp.clip(cnt_ref[:, j:j + 1], 1.0, None)
    z = lax.dot_general(msum, w_ref[...], (((1,), (1,)), ((), ())),
                        preferred_element_type=_f32)
    z_ref[...] = z

    @pl.when(i == 0)
    def _():
        st_ref[...] = jnp.zeros((2, D), _f32)

    st_ref[0:1, :] += jnp.sum(z, axis=0, keepdims=True)
    st_ref[1:2, :] += jnp.sum(z * z, axis=0, keepdims=True)


def _ftwa(acc, cnt3, w):
    return pl.pallas_call(
        _ftwa_body,
        grid=(_FTW_G,),
        in_specs=[pl.BlockSpec((3, 2, _FTW_BLK, D // 2), lambda i: (0, 0, i, 0)),
                  pl.BlockSpec((_FTW_BLK, 3), lambda i: (i, 0)),
                  pl.BlockSpec((D, D), lambda i: (0, 0))],
        out_specs=[pl.BlockSpec((_FTW_BLK, D), lambda i: (i, 0)),
                   pl.BlockSpec((2, D), lambda i: (0, 0))],
        out_shape=[jax.ShapeDtypeStruct((N_NODES, D), _f32),
                   jax.ShapeDtypeStruct((2, D), _f32)],
    )(acc, cnt3, w)


def _ftwb_body(h_ref, z_ref, st_ref, g_ref, be_ref, o_ref):
    z = z_ref[...]
    m = st_ref[0:1, :] / N_NODES
    var = st_ref[1:2, :] / N_NODES - m * m
    zn = (z - m) * lax.rsqrt(var + EPS) * g_ref[...] + be_ref[...]
    o_ref[...] = h_ref[...] + jnp.maximum(zn, 0.0)


def _ftwb(h, z, st, g, be):
    return pl.pallas_call(
        _ftwb_body,
        out_shape=jax.ShapeDtypeStruct((N_NODES, D), _f32),
    )(h, z, st, g, be)


def _head_body(hs_ref, ht_ref, bs_ref, bt_ref, y_ref,
               w1_ref, g1_ref, be1_ref, w2_ref, g2_ref, be2_ref, o_ref):
    def pool(h_ref, b_ref):
        b = b_ref[...]
        it = lax.broadcasted_iota(jnp.int32, (N_NODES, N_GRAPH), 1)
        oh = (b == it).astype(_f32)
        c = jnp.sum(oh, axis=0, keepdims=True)
        g = lax.dot_general(oh, h_ref[...], (((0,), (0,)), ((), ())),
                            preferred_element_type=_f32)
        return g / jnp.clip(c.T, 1.0, None)

    x = jnp.concatenate([pool(hs_ref, bs_ref), pool(ht_ref, bt_ref)], axis=1)
    z = lax.dot_general(x, w1_ref[...], (((1,), (1,)), ((), ())),
                        preferred_element_type=_f32)
    m = jnp.mean(z, axis=0, keepdims=True)
    v = jnp.mean(z * z, axis=0, keepdims=True) - m * m
    z = jnp.maximum((z - m) * lax.rsqrt(v + EPS) * g1_ref[...] + be1_ref[...], 0.0)
    p = lax.dot_general(z, w2_ref[...], (((1,), (1,)), ((), ())),
                        preferred_element_type=_f32)
    m = jnp.mean(p, axis=0, keepdims=True)
    v = jnp.mean(p * p, axis=0, keepdims=True) - m * m
    p = (p - m) * lax.rsqrt(v + EPS) * g2_ref[...] + be2_ref[...]
    pm = jnp.max(p, axis=1, keepdims=True)
    lse = jnp.log(jnp.sum(jnp.exp(p - pm), axis=1, keepdims=True)) + pm
    logp = p - lse
    yoh = (y_ref[...] == lax.broadcasted_iota(jnp.int32, (N_GRAPH, 2), 1))
    loss = -jnp.sum(jnp.where(yoh, logp, 0.0)) / N_GRAPH
    o_ref[...] = jnp.reshape(loss, (1, 1))


def _head(hs, ht, bs2, bt2, y2, w1, g1, be1, w2, g2, be2):
    return pl.pallas_call(
        _head_body,
        out_shape=jax.ShapeDtypeStruct((1, 1), _f32),
    )(hs, ht, bs2, bt2, y2, w1, g1, be1, w2, g2, be2)


def _cntred_body(p_ref, o_ref):
    o_ref[...] = jnp.sum(p_ref[...], axis=0, keepdims=True)


def _cntred(part2d):
    return pl.pallas_call(
        _cntred_body,
        out_shape=jax.ShapeDtypeStruct((1, 6 * NPAD), _f32),
    )(part2d)


_P1_C = 32
_P1_PER = 5120
_P1_CHUNKS = _P1_PER // _P1_C
_P2_C = 128
_P2_PER = 10240
_P2_CHUNKS = _P2_PER // _P2_C
NPAD = 10240
_ROWS_TEC = NPAD // NS
_RB = 128


def _sc_mesh():
    return plsc.VectorSubcoreMesh(core_axis_name="c", subcore_axis_name="s")


@functools.cache
def _pass1_kernel():
    return functools.partial(
        pl.kernel,
        out_type=[jax.ShapeDtypeStruct((N_WALKS, D3), _f32),
                  jax.ShapeDtypeStruct((NW, 1, D3), _f32)],
        mesh=_sc_mesh(),
        scratch_types=[pltpu.VMEM((_P1_C,), jnp.int32),
                       pltpu.VMEM((_P1_C,), jnp.int32),
                       pltpu.VMEM((_P1_C,), jnp.int32),
                       pltpu.VMEM((_P1_C, D3), _f32),
                       pltpu.VMEM((_P1_C, D3), _f32),
                       pltpu.VMEM((_P1_C, D3), _f32),
                       pltpu.VMEM((_P1_C, D3), _f32),
                       pltpu.VMEM((1, D3), _f32),
                       pltpu.SemaphoreType.DMA],
        compiler_params=pltpu.CompilerParams(needs_layout_passes=False),
    )(_pass1_body)


def _pass1(t, iflat):
    return _pass1_kernel()(t, iflat)


def _pass1_body(t_hbm, iflat_hbm, s_hbm, ssq_hbm,
                i0, i1, i2, b0, b1, b2, sb, ssq, sem):
    cid = lax.axis_index("c")
    sid = lax.axis_index("s")
    wid = sid * NC + cid
    zero16 = jnp.zeros((16,), _f32)
    for cc in range(D3 // 16):
        ssq[0, pl.ds(cc * 16, 16)] = zero16
    base0 = wid * _P1_PER

    def chunk(c, carry):
        base = base0 + c * _P1_C

        @pl.when(base < N_WALKS)
        def _():
            pltpu.sync_copy(iflat_hbm.at[pl.ds(base, _P1_C)], i0)
            pltpu.sync_copy(iflat_hbm.at[pl.ds(N_WALKS + base, _P1_C)], i1)
            pltpu.sync_copy(iflat_hbm.at[pl.ds(2 * N_WALKS + base, _P1_C)], i2)
            d0 = pltpu.async_copy(t_hbm.at[i0], b0, sem)
            d1 = pltpu.async_copy(t_hbm.at[i1], b1, sem)
            d2 = pltpu.async_copy(t_hbm.at[i2], b2, sem)
            d0.wait()
            d1.wait()
            d2.wait()

            def wbody(w, carry2):
                for cc in range(D3 // 16):
                    sl = pl.ds(cc * 16, 16)
                    v = b0[w, sl] + b1[w, sl] + b2[w, sl]
                    sb[w, sl] = v
                    plsc.addupdate(ssq.at[0, sl], v * v)
                return carry2

            lax.fori_loop(0, _P1_C, wbody, 0)
            pltpu.sync_copy(sb, s_hbm.at[pl.ds(base, _P1_C), :])

        return carry

    lax.fori_loop(0, _P1_CHUNKS, chunk, 0)
    pltpu.sync_copy(ssq, ssq_hbm.at[wid])


@functools.cache
def _pass2_kernel():
    return functools.partial(
        pl.kernel,
        out_type=jax.ShapeDtypeStruct((3, 2, NPAD, D // 2), _f32),
        mesh=_sc_mesh(),
        scratch_types=[pltpu.VMEM_SHARED((NPAD, D // 2), _f32),
                       pltpu.VMEM((_P2_C,), jnp.int32),
                       pltpu.VMEM((_P2_C, D // 2), _f32),
                       pltpu.VMEM((_RB, D // 2), _f32),
                       pltpu.VMEM((D // 2,), _f32),
                       pltpu.VMEM((D // 2,), _f32)],
        compiler_params=pltpu.CompilerParams(needs_layout_passes=False),
    )(_pass2_body)


def _pass2(s, rflat, alpha, beta):
    return _pass2_kernel()(s, rflat, alpha, beta)


def _pass2_body(s_hbm, idx_hbm, alpha_hbm, beta_hbm, acc_hbm,
                acc_s, ib, sb, tb, ab, bb):
    cid = lax.axis_index("c")
    sid = lax.axis_index("s")
    H = D // 2
    zero16 = jnp.zeros((16,), _f32)

    def zrow(w, carry):
        for q in range(H // 16):
            tb[w, pl.ds(q * 16, 16)] = zero16
        return carry

    lax.fori_loop(0, _RB, zrow, 0)

    for j in range(3):
        col = j * D + cid * H

        for i in range(_ROWS_TEC // _RB):
            pltpu.sync_copy(tb, acc_s.at[pl.ds(sid * _ROWS_TEC + i * _RB, _RB), :])

        pltpu.sync_copy(alpha_hbm.at[0, pl.ds(col, H)], ab)
        pltpu.sync_copy(beta_hbm.at[0, pl.ds(col, H)], bb)
        av = [ab[pl.ds(q * 16, 16)] for q in range(H // 16)]
        bv = [bb[pl.ds(q * 16, 16)] for q in range(H // 16)]
        plsc.subcore_barrier()

        base0 = sid * _P2_PER

        def chunk(c, carry):
            base = base0 + c * _P2_C

            @pl.when(base < N_WALKS)
            def _():
                pltpu.sync_copy(idx_hbm.at[pl.ds(j * N_WALKS + base, _P2_C)], ib)
                pltpu.sync_copy(s_hbm.at[pl.ds(base, _P2_C), pl.ds(col, H)], sb)

                def wbody(w, carry2):
                    for q in range(H // 16):
                        sl = pl.ds(q * 16, 16)
                        sb[w, sl] = jnp.maximum(sb[w, sl] * av[q] + bv[q], 0.0)
                    return carry2

                lax.fori_loop(0, _P2_C, wbody, 0)
                pltpu.sync_copy(sb, acc_s.at[ib], add=True)

            return carry

        lax.fori_loop(0, _P2_CHUNKS, chunk, 0)
        plsc.subcore_barrier()

        for i in range(_ROWS_TEC // _RB):
            r = sid * _ROWS_TEC + i * _RB
            pltpu.sync_copy(acc_s.at[pl.ds(r, _RB), :], sb)
            pltpu.sync_copy(sb, acc_hbm.at[j, cid, pl.ds(r, _RB), :])
        plsc.subcore_barrier()


_CNT_PER = 5120
_CNT_CHUNKS = _CNT_PER // _P2_C
_CNT_HALF = N_WALKS // 2


@functools.cache
def _counts_kernel():
    return functools.partial(
        pl.kernel,
        out_type=jax.ShapeDtypeStruct((NW * 6 * NPAD,), _f32),
        mesh=_sc_mesh(),
        scratch_types=[pltpu.VMEM((6 * NPAD,), _f32),
                       pltpu.VMEM((1, _P2_C), jnp.int32)],
        compiler_params=pltpu.CompilerParams(needs_layout_passes=False),
    )(_counts_body)


def _counts(idx6off):
    return _counts_kernel()(idx6off)


def _counts_body(idx6_hbm, part_hbm, cnt, ib):
    cid = lax.axis_index("c")
    sid = lax.axis_index("s")
    wid = sid * NC + cid
    zero16 = jnp.zeros((16,), _f32)
    one16 = jnp.ones((16,), _f32)

    def zrow(i, carry):
        cnt[pl.ds(i * 16, 16)] = zero16
        return carry

    lax.fori_loop(0, 6 * NPAD // 16, zrow, 0)

    for j in range(6):
        def chunk(c, carry):
            base = wid * _P1_PER + c * _P2_C

            @pl.when(base < N_WALKS)
            def _():
                pltpu.sync_copy(idx6_hbm.at[pl.ds(j * N_WALKS + base, _P2_C)], ib.at[0])
                for g in range(_P2_C // 16):
                    iv = ib[0, pl.ds(g * 16, 16)]
                    plsc.addupdate_scatter(cnt, [iv], one16)

            return carry

        lax.fori_loop(0, _P1_PER // _P2_C, chunk, 0)

    pltpu.sync_copy(cnt, part_hbm.at[pl.ds(wid * 6 * NPAD, 6 * NPAD)])


def kernel(x_s, x_t, term_walk_index_s, term_walk_index_t,
           x_s_batch, x_t_batch, y, params):
    P = params
    h0s = _embed(x_s, P["emb"])
    h0t = _embed(x_t, P["emb"])

    idx6 = jnp.concatenate([term_walk_index_s, term_walk_index_t], axis=0)
    idx6off = (idx6 + NPAD * jnp.arange(6, dtype=jnp.int32)[:, None]).reshape(-1)
    part = _counts(idx6off)
    red = _cntred(part.reshape(NW, 6 * NPAD))
    cnt6 = red.reshape(6, NPAD)[:, :N_NODES].T

    def side(h, twi, cnt3):
        idx3 = (twi * 3 + jnp.arange(3, dtype=jnp.int32)[:, None]).reshape(-1)
        for i in range(LAYERS):
            wcat_k = []
            for k in range(3):
                cols = [P[nm][i]["W"][:, k * D:(k + 1) * D].T
                        for nm in ("FT", "FM", "FB")]
                wcat_k.append(jnp.concatenate(cols, axis=1))
            wcat = jnp.concatenate(wcat_k, axis=1)
            gcat = jnp.concatenate(
                [P[nm][i]["g"] for nm in ("FT", "FM", "FB")]).reshape(1, D3)
            becat = jnp.concatenate(
                [P[nm][i]["be"] for nm in ("FT", "FM", "FB")]).reshape(1, D3)

            a, mean_s = _amat(h, wcat, cnt3)
            t = a.reshape(3 * N_NODES, D3)
            s, ssqp = _pass1(t, idx3)
            alpha, beta = _fin(ssqp.reshape(NW, D3), mean_s, gcat, becat)
            acc = _pass2(s, twi.reshape(-1), alpha, beta)
            z, st = _ftwa(acc, cnt3, P["FTW"][i]["W"])
            h = _ftwb(h, z, st,
                      P["FTW"][i]["g"].reshape(1, D),
                      P["FTW"][i]["be"].reshape(1, D))
        return h

    hs = side(h0s, term_walk_index_s, cnt6[:, 0:3])
    ht = side(h0t, term_walk_index_t, cnt6[:, 3:6])

    loss = _head(hs, ht,
                 x_s_batch.reshape(N_NODES, 1), x_t_batch.reshape(N_NODES, 1),
                 y.reshape(N_GRAPH, 1),
                 P["C1"]["W"], P["C1"]["g"].reshape(1, D),
                 P["C1"]["be"].reshape(1, D),
                 P["C2"]["W"], P["C2"]["g"].reshape(1, 2),
                 P["C2"]["be"].reshape(1, 2))
    return loss.reshape(())


# speedup vs baseline: 1.3882x; 1.3197x over previous
import functools

import jax
import jax.numpy as jnp
from jax import lax
from jax.experimental import pallas as pl
from jax.experimental.pallas import tpu as pltpu
from jax.experimental.pallas import tpu_sc as plsc

N_NODES = 10000
N_WALKS = 160000
N_IN = 128
D = 256
D3 = 3 * D
LAYERS = 2
N_GRAPH = 64
EPS = 1e-5

NC = 2
NS = 16
NW = NC * NS

_f32 = jnp.float32


def _embed_body(x_ref, emb_ref, o_ref):
    x = x_ref[...]
    mx = jnp.max(x, axis=1, keepdims=True)
    eq = x == mx
    it = lax.broadcasted_iota(jnp.int32, x.shape, 1)
    first_idx = jnp.min(jnp.where(eq, it, N_IN), axis=1, keepdims=True)
    oh = (it == first_idx).astype(_f32)
    o_ref[...] = jnp.dot(oh, emb_ref[...], preferred_element_type=_f32)


def _embed(x, emb):
    return pl.pallas_call(
        _embed_body,
        out_shape=jax.ShapeDtypeStruct((N_NODES, D), _f32),
    )(x, emb)


_AM_BLK = 2000
_AM_G = N_NODES // _AM_BLK


def _amat_body(h_ref, w_ref, cnt_ref, a_ref, mean_ref):
    i = pl.program_id(0)
    a = jnp.dot(h_ref[...], w_ref[...], preferred_element_type=_f32)
    a_ref[...] = a
    m = jnp.zeros((1, D3), _f32)
    for k in range(3):
        ck = cnt_ref[:, k:k + 1]
        m = m + lax.dot_general(ck, a[:, k * D3:(k + 1) * D3],
                                (((0,), (0,)), ((), ())),
                                preferred_element_type=_f32)

    @pl.when(i == 0)
    def _():
        mean_ref[...] = jnp.zeros((1, D3), _f32)

    mean_ref[...] += m / N_WALKS


def _amat(h, wcat, cnt3):
    return pl.pallas_call(
        _amat_body,
        grid=(_AM_G,),
        in_specs=[pl.BlockSpec((_AM_BLK, D), lambda i: (i, 0)),
                  pl.BlockSpec((D, 3 * D3), lambda i: (0, 0)),
                  pl.BlockSpec((_AM_BLK, 3), lambda i: (i, 0))],
        out_specs=[pl.BlockSpec((_AM_BLK, 3 * D3), lambda i: (i, 0)),
                   pl.BlockSpec((1, D3), lambda i: (0, 0))],
        out_shape=[jax.ShapeDtypeStruct((N_NODES, 3 * D3), _f32),
                   jax.ShapeDtypeStruct((1, D3), _f32)],
    )(h, wcat, cnt3)


def _fin_body(ssq_ref, mean_ref, g_ref, be_ref, a_ref, b_ref):
    ssq = jnp.sum(ssq_ref[...], axis=0, keepdims=True)
    mean = mean_ref[...]
    var = ssq / N_WALKS - mean * mean
    alpha = g_ref[...] * lax.rsqrt(var + EPS)
    a_ref[...] = alpha
    b_ref[...] = be_ref[...] - mean * alpha


def _fin(ssqp, mean, g, be):
    return pl.pallas_call(
        _fin_body,
        out_shape=[jax.ShapeDtypeStruct((1, D3), _f32),
                   jax.ShapeDtypeStruct((1, D3), _f32)],
    )(ssqp, mean, g, be)


_FTW_BLK = 2000
_FTW_G = N_NODES // _FTW_BLK


def _ftwa_body(acc_ref, cnt_ref, w_ref, z_ref, st_ref):
    i = pl.program_id(0)
    msum = jnp.zeros((_FTW_BLK, D), _f32)
    for j in range(3):
        aj = jnp.concatenate([acc_ref[j, 0], acc_ref[j, 1]], axis=1)
        msum = msum + aj / jnp.clip(cnt_ref[:, j:j + 1], 1.0, None)
    z = lax.dot_general(msum, w_ref[...], (((1,), (1,)), ((), ())),
                        preferred_element_type=_f32)
    z_ref[...] = z

    @pl.when(i == 0)
    def _():
        st_ref[...] = jnp.zeros((2, D), _f32)

    st_ref[0:1, :] += jnp.sum(z, axis=0, keepdims=True)
    st_ref[1:2, :] += jnp.sum(z * z, axis=0, keepdims=True)


def _ftwa(acc, cnt3, w):
    return pl.pallas_call(
        _ftwa_body,
        grid=(_FTW_G,),
        in_specs=[pl.BlockSpec((3, 2, _FTW_BLK, D // 2), lambda i: (0, 0, i, 0)),
                  pl.BlockSpec((_FTW_BLK, 3), lambda i: (i, 0)),
                  pl.BlockSpec((D, D), lambda i: (0, 0))],
        out_specs=[pl.BlockSpec((_FTW_BLK, D), lambda i: (i, 0)),
                   pl.BlockSpec((2, D), lambda i: (0, 0))],
        out_shape=[jax.ShapeDtypeStruct((N_NODES, D), _f32),
                   jax.ShapeDtypeStruct((2, D), _f32)],
    )(acc, cnt3, w)


def _ftwb_body(h_ref, z_ref, st_ref, g_ref, be_ref, o_ref):
    z = z_ref[...]
    m = st_ref[0:1, :] / N_NODES
    var = st_ref[1:2, :] / N_NODES - m * m
    zn = (z - m) * lax.rsqrt(var + EPS) * g_ref[...] + be_ref[...]
    o_ref[...] = h_ref[...] + jnp.maximum(zn, 0.0)


def _ftwb(h, z, st, g, be):
    return pl.pallas_call(
        _ftwb_body,
        out_shape=jax.ShapeDtypeStruct((N_NODES, D), _f32),
    )(h, z, st, g, be)


def _head_body(hs_ref, ht_ref, bs_ref, bt_ref, y_ref,
               w1_ref, g1_ref, be1_ref, w2_ref, g2_ref, be2_ref, o_ref):
    def pool(h_ref, b_ref):
        b = b_ref[...]
        it = lax.broadcasted_iota(jnp.int32, (N_NODES, N_GRAPH), 1)
        oh = (b == it).astype(_f32)
        c = jnp.sum(oh, axis=0, keepdims=True)
        g = lax.dot_general(oh, h_ref[...], (((0,), (0,)), ((), ())),
                            preferred_element_type=_f32)
        return g / jnp.clip(c.T, 1.0, None)

    x = jnp.concatenate([pool(hs_ref, bs_ref), pool(ht_ref, bt_ref)], axis=1)
    z = lax.dot_general(x, w1_ref[...], (((1,), (1,)), ((), ())),
                        preferred_element_type=_f32)
    m = jnp.mean(z, axis=0, keepdims=True)
    v = jnp.mean(z * z, axis=0, keepdims=True) - m * m
    z = jnp.maximum((z - m) * lax.rsqrt(v + EPS) * g1_ref[...] + be1_ref[...], 0.0)
    p = lax.dot_general(z, w2_ref[...], (((1,), (1,)), ((), ())),
                        preferred_element_type=_f32)
    m = jnp.mean(p, axis=0, keepdims=True)
    v = jnp.mean(p * p, axis=0, keepdims=True) - m * m
    p = (p - m) * lax.rsqrt(v + EPS) * g2_ref[...] + be2_ref[...]
    pm = jnp.max(p, axis=1, keepdims=True)
    lse = jnp.log(jnp.sum(jnp.exp(p - pm), axis=1, keepdims=True)) + pm
    logp = p - lse
    yoh = (y_ref[...] == lax.broadcasted_iota(jnp.int32, (N_GRAPH, 2), 1))
    loss = -jnp.sum(jnp.where(yoh, logp, 0.0)) / N_GRAPH
    o_ref[...] = jnp.reshape(loss, (1, 1))


def _head(hs, ht, bs2, bt2, y2, w1, g1, be1, w2, g2, be2):
    return pl.pallas_call(
        _head_body,
        out_shape=jax.ShapeDtypeStruct((1, 1), _f32),
    )(hs, ht, bs2, bt2, y2, w1, g1, be1, w2, g2, be2)


def _cntred_body(p_ref, o_ref):
    o_ref[...] = jnp.sum(p_ref[...], axis=0, keepdims=True)


def _cntred(part2d):
    return pl.pallas_call(
        _cntred_body,
        out_shape=jax.ShapeDtypeStruct((1, 6 * NPAD), _f32),
    )(part2d)


_P1_C = 32
_P1_PER = 5120
_P1_CHUNKS = _P1_PER // _P1_C
_P2_C = 128
_P2_PER = 10240
_P2_CHUNKS = _P2_PER // _P2_C
NPAD = 10240
_ROWS_TEC = NPAD // NS
_RB = 128


def _sc_mesh():
    return plsc.VectorSubcoreMesh(core_axis_name="c", subcore_axis_name="s")


N_PADW = 163840


_P1C = 16
_P1_CH = _P1_PER // _P1C


@functools.cache
def _pass1_kernel():
    gb = [pltpu.VMEM((_P1C, D3), _f32)] * 6
    return functools.partial(
        pl.kernel,
        out_type=[jax.ShapeDtypeStruct((N_WALKS, D3), _f32),
                  jax.ShapeDtypeStruct((NW, 1, D3), _f32)],
        mesh=_sc_mesh(),
        scratch_types=[pltpu.VMEM((_P1_PER,), jnp.int32),
                       pltpu.VMEM((_P1_PER,), jnp.int32),
                       pltpu.VMEM((_P1_PER,), jnp.int32),
                       *gb,
                       pltpu.VMEM((_P1C, D3), _f32),
                       pltpu.VMEM((_P1C, D3), _f32),
                       pltpu.VMEM((1, D3), _f32),
                       pltpu.SemaphoreType.DMA,
                       pltpu.SemaphoreType.DMA,
                       pltpu.SemaphoreType.DMA,
                       pltpu.SemaphoreType.DMA],
        compiler_params=pltpu.CompilerParams(needs_layout_passes=False),
    )(_pass1_body)


def _pass1(t, iflat):
    return _pass1_kernel()(t, iflat)


def _pass1_body(t_hbm, iflat_hbm, s_hbm, ssq_hbm,
                i0a, i1a, i2a, b00, b01, b02, b10, b11, b12,
                sb0, sb1, ssq, g0sem, g1sem, s0sem, s1sem):
    cid = lax.axis_index("c")
    sid = lax.axis_index("s")
    wid = sid * NC + cid
    zero16 = jnp.zeros((16,), _f32)
    for cc in range(D3 // 16):
        ssq[0, pl.ds(cc * 16, 16)] = zero16
    base0 = wid * _P1_PER

    pltpu.sync_copy(iflat_hbm.at[pl.ds(base0, _P1_PER)], i0a)
    pltpu.sync_copy(iflat_hbm.at[pl.ds(N_PADW + base0, _P1_PER)], i1a)
    pltpu.sync_copy(iflat_hbm.at[pl.ds(2 * N_PADW + base0, _P1_PER)], i2a)

    slots = ((b00, b01, b02, sb0, g0sem, s0sem),
             (b10, b11, b12, sb1, g1sem, s1sem))

    def fire(c, slot):
        b0, b1, b2, _, gsem, _ = slot

        @pl.when(base0 + c * _P1C < N_WALKS)
        def _():
            sl = pl.ds(c * _P1C, _P1C)
            pltpu.async_copy(t_hbm.at[i0a.at[sl]], b0, gsem)
            pltpu.async_copy(t_hbm.at[i1a.at[sl]], b1, gsem)
            pltpu.async_copy(t_hbm.at[i2a.at[sl]], b2, gsem)

    def work(c, slot, first):
        b0, b1, b2, sb, gsem, ssem = slot
        base = base0 + c * _P1C

        @pl.when(base < N_WALKS)
        def _():
            dummy = t_hbm.at[pl.ds(0, _P1C), :]
            for b in (b0, b1, b2):
                pltpu.make_async_copy(dummy, b, gsem).wait()
            @pl.when(jnp.logical_not(first))
            def _():
                pltpu.make_async_copy(sb, s_hbm.at[pl.ds(0, _P1C), :], ssem).wait()

            def wbody(w, carry2):
                for cc in range(D3 // 16):
                    sl = pl.ds(cc * 16, 16)
                    v = b0[w, sl] + b1[w, sl] + b2[w, sl]
                    sb[w, sl] = v
                    plsc.addupdate(ssq.at[0, sl], v * v)
                return carry2

            lax.fori_loop(0, _P1C, wbody, 0)
            pltpu.async_copy(sb, s_hbm.at[pl.ds(base, _P1C), :], ssem)

    fire(0, slots[0])
    fire(1, slots[1])

    def pair(p, carry):
        c0 = 2 * p
        work(c0, slots[0], p == 0)
        fire(c0 + 2, slots[0])
        work(c0 + 1, slots[1], p == 0)
        fire(c0 + 3, slots[1])
        return carry

    lax.fori_loop(0, _P1_CH // 2 - 1, pair, 0)
    work(_P1_CH - 2, slots[0], False)
    work(_P1_CH - 1, slots[1], False)
    for slot in slots:
        sb, ssem = slot[3], slot[5]

        @pl.when(base0 < N_WALKS)
        def _():
            pltpu.make_async_copy(sb, s_hbm.at[pl.ds(0, _P1C), :], ssem).wait()

    pltpu.sync_copy(ssq, ssq_hbm.at[wid])


@functools.cache
def _pass2_kernel():
    return functools.partial(
        pl.kernel,
        out_type=jax.ShapeDtypeStruct((3, 2, NPAD, D // 2), _f32),
        mesh=_sc_mesh(),
        scratch_types=[pltpu.VMEM_SHARED((NPAD, D // 2), _f32),
                       pltpu.VMEM((_P2_C,), jnp.int32),
                       pltpu.VMEM((_P2_C, D // 2), _f32),
                       pltpu.VMEM((_RB, D // 2), _f32),
                       pltpu.VMEM((D // 2,), _f32),
                       pltpu.VMEM((D // 2,), _f32)],
        compiler_params=pltpu.CompilerParams(needs_layout_passes=False),
    )(_pass2_body)


def _pass2(s, rflat, alpha, beta):
    return _pass2_kernel()(s, rflat, alpha, beta)


def _pass2_body(s_hbm, idx_hbm, alpha_hbm, beta_hbm, acc_hbm,
                acc_s, ib, sb, tb, ab, bb):
    cid = lax.axis_index("c")
    sid = lax.axis_index("s")
    H = D // 2
    zero16 = jnp.zeros((16,), _f32)

    def zrow(w, carry):
        for q in range(H // 16):
            tb[w, pl.ds(q * 16, 16)] = zero16
        return carry

    lax.fori_loop(0, _RB, zrow, 0)

    for j in range(3):
        col = j * D + cid * H

        for i in range(_ROWS_TEC // _RB):
            pltpu.sync_copy(tb, acc_s.at[pl.ds(sid * _ROWS_TEC + i * _RB, _RB), :])

        pltpu.sync_copy(alpha_hbm.at[0, pl.ds(col, H)], ab)
        pltpu.sync_copy(beta_hbm.at[0, pl.ds(col, H)], bb)
        av = [ab[pl.ds(q * 16, 16)] for q in range(H // 16)]
        bv = [bb[pl.ds(q * 16, 16)] for q in range(H // 16)]
        plsc.subcore_barrier()

        base0 = sid * _P2_PER

        def chunk(c, carry):
            base = base0 + c * _P2_C

            @pl.when(base < N_WALKS)
            def _():
                pltpu.sync_copy(idx_hbm.at[pl.ds(j * N_WALKS + base, _P2_C)], ib)
                pltpu.sync_copy(s_hbm.at[pl.ds(base, _P2_C), pl.ds(col, H)], sb)

                def wbody(w, carry2):
                    for q in range(H // 16):
                        sl = pl.ds(q * 16, 16)
                        sb[w, sl] = jnp.maximum(sb[w, sl] * av[q] + bv[q], 0.0)
                    return carry2

                lax.fori_loop(0, _P2_C, wbody, 0)
                pltpu.sync_copy(sb, acc_s.at[ib], add=True)

            return carry

        lax.fori_loop(0, _P2_CHUNKS, chunk, 0)
        plsc.subcore_barrier()

        for i in range(_ROWS_TEC // _RB):
            r = sid * _ROWS_TEC + i * _RB
            pltpu.sync_copy(acc_s.at[pl.ds(r, _RB), :], sb)
            pltpu.sync_copy(sb, acc_hbm.at[j, cid, pl.ds(r, _RB), :])
        plsc.subcore_barrier()


_CNT_PER = 5120
_CNT_CHUNKS = _CNT_PER // _P2_C
_CNT_HALF = N_WALKS // 2


@functools.cache
def _counts_kernel():
    return functools.partial(
        pl.kernel,
        out_type=jax.ShapeDtypeStruct((NW * 6 * NPAD,), _f32),
        mesh=_sc_mesh(),
        scratch_types=[pltpu.VMEM((6 * NPAD,), _f32),
                       pltpu.VMEM((1, _P2_C), jnp.int32)],
        compiler_params=pltpu.CompilerParams(needs_layout_passes=False),
    )(_counts_body)


def _counts(idx6off):
    return _counts_kernel()(idx6off)


def _counts_body(idx6_hbm, part_hbm, cnt, ib):
    cid = lax.axis_index("c")
    sid = lax.axis_index("s")
    wid = sid * NC + cid
    zero16 = jnp.zeros((16,), _f32)
    one16 = jnp.ones((16,), _f32)

    def zrow(i, carry):
        cnt[pl.ds(i * 16, 16)] = zero16
        return carry

    lax.fori_loop(0, 6 * NPAD // 16, zrow, 0)

    for j in range(6):
        def chunk(c, carry):
            base = wid * _P1_PER + c * _P2_C

            @pl.when(base < N_WALKS)
            def _():
                pltpu.sync_copy(idx6_hbm.at[pl.ds(j * N_WALKS + base, _P2_C)], ib.at[0])
                for g in range(_P2_C // 16):
                    iv = ib[0, pl.ds(g * 16, 16)]
                    plsc.addupdate_scatter(cnt, [iv], one16)

            return carry

        lax.fori_loop(0, _P1_PER // _P2_C, chunk, 0)

    pltpu.sync_copy(cnt, part_hbm.at[pl.ds(wid * 6 * NPAD, 6 * NPAD)])


def kernel(x_s, x_t, term_walk_index_s, term_walk_index_t,
           x_s_batch, x_t_batch, y, params):
    P = params
    h0s = _embed(x_s, P["emb"])
    h0t = _embed(x_t, P["emb"])

    idx6 = jnp.concatenate([term_walk_index_s, term_walk_index_t], axis=0)
    idx6off = (idx6 + NPAD * jnp.arange(6, dtype=jnp.int32)[:, None]).reshape(-1)
    part = _counts(idx6off)
    red = _cntred(part.reshape(NW, 6 * NPAD))
    cnt6 = red.reshape(6, NPAD)[:, :N_NODES].T

    def side(h, twi, cnt3):
        i3 = twi * 3 + jnp.arange(3, dtype=jnp.int32)[:, None]
        idx3 = jnp.zeros((3, N_PADW), jnp.int32).at[:, :N_WALKS].set(i3).reshape(-1)
        for i in range(LAYERS):
            wcat_k = []
            for k in range(3):
                cols = [P[nm][i]["W"][:, k * D:(k + 1) * D].T
                        for nm in ("FT", "FM", "FB")]
                wcat_k.append(jnp.concatenate(cols, axis=1))
            wcat = jnp.concatenate(wcat_k, axis=1)
            gcat = jnp.concatenate(
                [P[nm][i]["g"] for nm in ("FT", "FM", "FB")]).reshape(1, D3)
            becat = jnp.concatenate(
                [P[nm][i]["be"] for nm in ("FT", "FM", "FB")]).reshape(1, D3)

            a, mean_s = _amat(h, wcat, cnt3)
            t = a.reshape(3 * N_NODES, D3)
            s, ssqp = _pass1(t, idx3)
            alpha, beta = _fin(ssqp.reshape(NW, D3), mean_s, gcat, becat)
            acc = _pass2(s, twi.reshape(-1), alpha, beta)
            z, st = _ftwa(acc, cnt3, P["FTW"][i]["W"])
            h = _ftwb(h, z, st,
                      P["FTW"][i]["g"].reshape(1, D),
                      P["FTW"][i]["be"].reshape(1, D))
        return h

    hs = side(h0s, term_walk_index_s, cnt6[:, 0:3])
    ht = side(h0t, term_walk_index_t, cnt6[:, 3:6])

    loss = _head(hs, ht,
                 x_s_batch.reshape(N_NODES, 1), x_t_batch.reshape(N_NODES, 1),
                 y.reshape(N_GRAPH, 1),
                 P["C1"]["W"], P["C1"]["g"].reshape(1, D),
                 P["C1"]["be"].reshape(1, D),
                 P["C2"]["W"], P["C2"]["g"].reshape(1, 2),
                 P["C2"]["be"].reshape(1, 2))
    return loss.reshape(())
